# baseline (speedup 1.0000x reference)
import functools

import numpy as np
import jax
import jax.numpy as jnp
from jax import lax
from jax.experimental import pallas as pl
from jax.experimental.pallas import tpu as pltpu
from jax.experimental.pallas import tpu_sc as plsc

_B, _M, _D, _E, _T = 4096, 6, 128, 8, 60
_H1, _H2, _OUT = 256, 256, _T * 2
_S1, _S2 = 128, 64


def _gelu(x):
    return 0.5 * x * (1.0 + lax.erf(x * jnp.float32(0.7071067811865476)))


_SQRT_HALF = np.float32(0.7071067811865476)
_ERF_COEFFS = [np.float32(c) for c in
               (7.85386146e-05, -0.000801019371, 0.00518832775,
                -0.0268538129, 0.112835854, -0.37612626, 1.12837911)]


def _gelu_router(x):
    a = -x * _SQRT_HALF
    t = a * a
    p = t * _ERF_COEFFS[0]
    for c in _ERF_COEFFS[1:-1]:
        p = (p + c) * t
    p = p + _ERF_COEFFS[-1]
    small = 1.0 - a * p
    big = 1.0 - lax.erf(a)
    erfc = jnp.where(jnp.abs(a) < 1.0, small, big)
    return 0.5 * x * erfc


def _router_body(x_ref, rw1_ref, rb1_ref, rw2_ref, rb2_ref, rw3_ref, rb3_ref,
                 logits_ref, probs_ref, aux_ref):
    x = x_ref[...]
    ctx = x[:, 0, :]
    for i in range(1, _M):
        ctx = ctx + x[:, i, :]
    ctx = ctx / jnp.float32(_M)
    h = _gelu_router(jnp.dot(ctx, rw1_ref[...]) + rb1_ref[...])
    h = _gelu_router(jnp.dot(h, rw2_ref[...]) + rb2_ref[...])
    logits = jnp.dot(h, rw3_ref[...]) + rb3_ref[...]
    logits_ref[...] = logits
    probs = jax.nn.softmax(logits, axis=-1)
    probs_ref[...] = probs
    avg = jnp.mean(probs, axis=0, keepdims=True)
    entropy = -jnp.sum(avg * jnp.log(avg + 1e-8))
    l2 = jnp.mean((avg - 1.0 / _E) ** 2)
    aux_ref[...] = (-entropy * 0.01 + 0.01 * l2).reshape(1, 1)


def _router(mode_features, rW1, rb1, rW2, rb2, rW3, rb3):
    return pl.pallas_call(
        _router_body,
        out_shape=[
            jax.ShapeDtypeStruct((_B, _E), jnp.float32),
            jax.ShapeDtypeStruct((_B, _E), jnp.float32),
            jax.ShapeDtypeStruct((1, 1), jnp.float32),
        ],
    )(mode_features, rW1, rb1.reshape(1, -1), rW2, rb2.reshape(1, -1),
      rW3, rb3.reshape(1, -1))


_NC, _NS, _L = 2, 16, 16
_NW = _NC * _NS
_CHUNK = _B // _NW
_NG = _CHUNK // _L


def _gates_sc_body(logitsT_hbm, gatesT_hbm, lg_v, gt_v):
    c = lax.axis_index("c")
    s = lax.axis_index("s")
    base = (s * _NC + c) * _CHUNK
    pltpu.sync_copy(logitsT_hbm.at[:, pl.ds(base, _CHUNK)], lg_v)
    mask = jnp.int32(0x7FFFFFFF)
    sent = jnp.int32(-2 ** 31)
    for g in range(_NG):
        ls = [lg_v[e, pl.ds(g * _L, _L)] for e in range(_E)]
        ks = []
        for e in range(_E):
            b = lax.bitcast_convert_type(ls[e], jnp.int32)
            ks.append(jnp.where(b < 0, b ^ mask, b))
        k1 = ks[0]
        for e in range(1, _E):
            k1 = jnp.maximum(k1, ks[e])
        idx1 = jnp.zeros((_L,), jnp.int32)
        for e in range(_E - 1, -1, -1):
            idx1 = jnp.where(ks[e] == k1, jnp.int32(e), idx1)
        k2 = jnp.full((_L,), sent, jnp.int32)
        for e in range(_E):
            ke = jnp.where(idx1 == e, jnp.full((_L,), sent, jnp.int32), ks[e])
            k2 = jnp.maximum(k2, ke)
        idx2 = jnp.zeros((_L,), jnp.int32)
        for e in range(_E - 1, -1, -1):
            hit = (ks[e] == k2) & (idx1 != e)
            idx2 = jnp.where(hit, jnp.int32(e), idx2)
        m1 = lax.bitcast_convert_type(jnp.where(k1 < 0, k1 ^ mask, k1), jnp.float32)
        m2 = lax.bitcast_convert_type(jnp.where(k2 < 0, k2 ^ mask, k2), jnp.float32)
        t = jnp.exp(m2 - m1)
        den = 1.0 + t
        p1 = 1.0 / den
        p2 = t / den
        zero = jnp.zeros((_L,), jnp.float32)
        for e in range(_E):
            ge = jnp.where(idx1 == e, p1, jnp.where(idx2 == e, p2, zero))
            gt_v[e, pl.ds(g * _L, _L)] = ge
    pltpu.sync_copy(gt_v, gatesT_hbm.at[:, pl.ds(base, _CHUNK)])


def _gates(logits):
    gatesT = pl.kernel(
        _gates_sc_body,
        out_type=jax.ShapeDtypeStruct((_E, _B), jnp.float32),
        mesh=plsc.VectorSubcoreMesh(core_axis_name="c", subcore_axis_name="s",
                                    num_cores=_NC, num_subcores=_NS),
        scratch_types=[
            pltpu.VMEM((_E, _CHUNK), jnp.float32),
            pltpu.VMEM((_E, _CHUNK), jnp.float32),
        ],
    )(logits.T)
    return gatesT.T


_BLK = 512
_W1N = _H1 + _S1


def _experts_body(x_ref, g_ref, w1_ref, wt2_ref, wt3_ref, ws2_ref,
                  ws3_ref, bt1_ref, bt2_ref, bt3_ref, bs1_ref, bs2_ref,
                  bs3_ref, traj_ref, sc_ref):
    ie = pl.program_id(1)
    bf = jnp.bfloat16
    f32 = jnp.float32
    xb = x_ref[...].reshape(_BLK * _M, _D)
    h0 = jnp.dot(xb, w1_ref[0], preferred_element_type=f32)
    h = _gelu(h0[:, :_H1] + bt1_ref[0])
    s = _gelu(h0[:, _H1:] + bs1_ref[0])
    h = _gelu(jnp.dot(h.astype(bf), wt2_ref[0],
                      preferred_element_type=f32) + bt2_ref[0])
    tr = jnp.dot(h.astype(bf), wt3_ref[0],
                 preferred_element_type=f32) + bt3_ref[0]
    s = _gelu(jnp.dot(s.astype(bf), ws2_ref[0],
                      preferred_element_type=f32) + bs2_ref[0])
    sc = jnp.sum(s * ws3_ref[0], axis=1, keepdims=True) + bs3_ref[0]
    lane = lax.broadcasted_iota(jnp.int32, (1, _E), 1)
    w = jnp.sum(jnp.where(lane == ie, g_ref[...], 0.0), axis=1,
                keepdims=True)
    w3 = w[:, :, None]
    tr3 = tr.reshape(_BLK, _M, _OUT) * w3
    sc3 = sc.reshape(_BLK, _M, 1) * w3

    @pl.when(ie == 0)
    def _init():
        traj_ref[...] = jnp.zeros_like(traj_ref)
        sc_ref[...] = jnp.zeros_like(sc_ref)

    traj_ref[...] += tr3
    sc_ref[...] += sc3


def _experts(xb16, gates, w1cat, wt2b, wt3b, ws2b, ws3t,
             bt1, bt2, bt3, bs1, bs2, bs3):
    bt1r = bt1.reshape(_E, 1, _H1)
    bt2r = bt2.reshape(_E, 1, _H2)
    bt3r = bt3.reshape(_E, 1, _OUT)
    bs1r = bs1.reshape(_E, 1, _S1)
    bs2r = bs2.reshape(_E, 1, _S2)
    bs3r = bs3.reshape(_E, 1, 1)
    pere = lambda a, b: pl.BlockSpec((1, a, b), lambda i, e: (e, 0, 0))
    return pl.pallas_call(
        _experts_body,
        grid=(_B // _BLK, _E),
        in_specs=[
            pl.BlockSpec((_BLK, _M, _D), lambda i, e: (i, 0, 0)),
            pl.BlockSpec((_BLK, _E), lambda i, e: (i, 0)),
            pere(_D, _W1N),
            pere(_H1, _H2),
            pere(_H2, _OUT),
            pere(_S1, _S2),
            pere(1, _S2),
            pere(1, _H1),
            pere(1, _H2),
            pere(1, _OUT),
            pere(1, _S1),
            pere(1, _S2),
            pere(1, 1),
        ],
        out_specs=[
            pl.BlockSpec((_BLK, _M, _OUT), lambda i, e: (i, 0, 0)),
            pl.BlockSpec((_BLK, _M, 1), lambda i, e: (i, 0, 0)),
        ],
        out_shape=[
            jax.ShapeDtypeStruct((_B, _M, _OUT), jnp.float32),
            jax.ShapeDtypeStruct((_B, _M, 1), jnp.float32),
        ],
        compiler_params=pltpu.CompilerParams(
            dimension_semantics=("arbitrary", "arbitrary")),
    )(xb16, gates, w1cat, wt2b, wt3b, ws2b, ws3t,
      bt1r, bt2r, bt3r, bs1r, bs2r, bs3r)


def kernel(mode_features, rW1, rb1, rW2, rb2, rW3, rb3, Wt1, bt1, Wt2, bt2,
           Wt3, bt3, Ws1, bs1, Ws2, bs2, Ws3, bs3):
    logits, router_probs, aux = _router(mode_features, rW1, rb1, rW2, rb2,
                                        rW3, rb3)
    gates = _gates(logits)
    bf = jnp.bfloat16
    xb16 = mode_features.astype(bf)
    w1cat = jnp.concatenate([Wt1, Ws1], axis=2).astype(bf)
    ws3t = jnp.swapaxes(Ws3, 1, 2)
    traj, sc = _experts(xb16, gates, w1cat, Wt2.astype(bf), Wt3.astype(bf),
                        Ws2.astype(bf), ws3t, bt1, bt2, bt3, bs1, bs2, bs3)
    trajectories = traj.reshape(_B, _M, _T, 2)
    scores = sc.reshape(_B, _M)
    return trajectories, scores, aux.reshape(()), router_probs


# speedup vs baseline: 1.0448x; 1.0448x over previous
import functools

import numpy as np
import jax
import jax.numpy as jnp
from jax import lax
from jax.experimental import pallas as pl
from jax.experimental.pallas import tpu as pltpu
from jax.experimental.pallas import tpu_sc as plsc

_B, _M, _D, _E, _T = 4096, 6, 128, 8, 60
_H1, _H2, _OUT = 256, 256, _T * 2
_S1, _S2 = 128, 64


def _gelu(x):
    return 0.5 * x * (1.0 + lax.erf(x * jnp.float32(0.7071067811865476)))


_SQRT_HALF = np.float32(0.7071067811865476)
_ERF_COEFFS = [np.float32(c) for c in
               (7.85386146e-05, -0.000801019371, 0.00518832775,
                -0.0268538129, 0.112835854, -0.37612626, 1.12837911)]


def _gelu_router(x):
    a = -x * _SQRT_HALF
    t = a * a
    p = t * _ERF_COEFFS[0]
    for c in _ERF_COEFFS[1:-1]:
        p = (p + c) * t
    p = p + _ERF_COEFFS[-1]
    small = 1.0 - a * p
    big = 1.0 - lax.erf(a)
    erfc = jnp.where(jnp.abs(a) < 1.0, small, big)
    return 0.5 * x * erfc


def _router_body(x_ref, rw1_ref, rb1_ref, rw2_ref, rb2_ref, rw3_ref, rb3_ref,
                 logits_ref, probs_ref, aux_ref):
    x = x_ref[...]
    ctx = x[:, 0, :]
    for i in range(1, _M):
        ctx = ctx + x[:, i, :]
    ctx = ctx / jnp.float32(_M)
    h = _gelu_router(jnp.dot(ctx, rw1_ref[...]) + rb1_ref[...])
    h = _gelu_router(jnp.dot(h, rw2_ref[...]) + rb2_ref[...])
    logits = jnp.dot(h, rw3_ref[...]) + rb3_ref[...]
    logits_ref[...] = logits
    probs = jax.nn.softmax(logits, axis=-1)
    probs_ref[...] = probs
    avg = jnp.mean(probs, axis=0, keepdims=True)
    entropy = -jnp.sum(avg * jnp.log(avg + 1e-8))
    l2 = jnp.mean((avg - 1.0 / _E) ** 2)
    aux_ref[...] = (-entropy * 0.01 + 0.01 * l2).reshape(1, 1)


def _router(mode_features, rW1, rb1, rW2, rb2, rW3, rb3):
    return pl.pallas_call(
        _router_body,
        out_shape=[
            jax.ShapeDtypeStruct((_B, _E), jnp.float32),
            jax.ShapeDtypeStruct((_B, _E), jnp.float32),
            jax.ShapeDtypeStruct((1, 1), jnp.float32),
        ],
    )(mode_features, rW1, rb1.reshape(1, -1), rW2, rb2.reshape(1, -1),
      rW3, rb3.reshape(1, -1))


_NC, _NS, _L = 2, 16, 16
_NW = _NC * _NS
_CHUNK = _B // _NW
_NG = _CHUNK // _L


def _gates_sc_body(logitsT_hbm, gatesT_hbm, lg_v, gt_v):
    c = lax.axis_index("c")
    s = lax.axis_index("s")
    base = (s * _NC + c) * _CHUNK
    pltpu.sync_copy(logitsT_hbm.at[:, pl.ds(base, _CHUNK)], lg_v)
    mask = jnp.int32(0x7FFFFFFF)
    sent = jnp.int32(-2 ** 31)
    for g in range(_NG):
        ls = [lg_v[e, pl.ds(g * _L, _L)] for e in range(_E)]
        ks = []
        for e in range(_E):
            b = lax.bitcast_convert_type(ls[e], jnp.int32)
            ks.append(jnp.where(b < 0, b ^ mask, b))
        k1 = ks[0]
        for e in range(1, _E):
            k1 = jnp.maximum(k1, ks[e])
        idx1 = jnp.zeros((_L,), jnp.int32)
        for e in range(_E - 1, -1, -1):
            idx1 = jnp.where(ks[e] == k1, jnp.int32(e), idx1)
        k2 = jnp.full((_L,), sent, jnp.int32)
        for e in range(_E):
            ke = jnp.where(idx1 == e, jnp.full((_L,), sent, jnp.int32), ks[e])
            k2 = jnp.maximum(k2, ke)
        idx2 = jnp.zeros((_L,), jnp.int32)
        for e in range(_E - 1, -1, -1):
            hit = (ks[e] == k2) & (idx1 != e)
            idx2 = jnp.where(hit, jnp.int32(e), idx2)
        m1 = lax.bitcast_convert_type(jnp.where(k1 < 0, k1 ^ mask, k1), jnp.float32)
        m2 = lax.bitcast_convert_type(jnp.where(k2 < 0, k2 ^ mask, k2), jnp.float32)
        t = jnp.exp(m2 - m1)
        den = 1.0 + t
        p1 = 1.0 / den
        p2 = t / den
        zero = jnp.zeros((_L,), jnp.float32)
        for e in range(_E):
            ge = jnp.where(idx1 == e, p1, jnp.where(idx2 == e, p2, zero))
            gt_v[e, pl.ds(g * _L, _L)] = ge
    pltpu.sync_copy(gt_v, gatesT_hbm.at[:, pl.ds(base, _CHUNK)])


def _gates(logits):
    gatesT = pl.kernel(
        _gates_sc_body,
        out_type=jax.ShapeDtypeStruct((_E, _B), jnp.float32),
        mesh=plsc.VectorSubcoreMesh(core_axis_name="c", subcore_axis_name="s",
                                    num_cores=_NC, num_subcores=_NS),
        scratch_types=[
            pltpu.VMEM((_E, _CHUNK), jnp.float32),
            pltpu.VMEM((_E, _CHUNK), jnp.float32),
        ],
    )(logits.T)
    return gatesT.T


_BLK = 512
_W1N = _H1 + _S1


def _experts_body(x_ref, g_ref, w1_ref, wt2_ref, wt3_ref, ws2_ref,
                  ws3_ref, bt1_ref, bt2_ref, bt3_ref, bs1_ref, bs2_ref,
                  bs3_ref, traj_ref, sc_ref):
    ie = pl.program_id(1)
    bf = jnp.bfloat16
    f32 = jnp.float32
    xb = x_ref[...].reshape(_BLK * _M, _D).astype(bf)
    h0 = jnp.dot(xb, w1_ref[0], preferred_element_type=f32)
    h = _gelu(h0[:, :_H1] + bt1_ref[0])
    s = _gelu(h0[:, _H1:] + bs1_ref[0])
    h = _gelu(jnp.dot(h.astype(bf), wt2_ref[0],
                      preferred_element_type=f32) + bt2_ref[0])
    tr = jnp.dot(h.astype(bf), wt3_ref[0],
                 preferred_element_type=f32) + bt3_ref[0]
    s = _gelu(jnp.dot(s.astype(bf), ws2_ref[0],
                      preferred_element_type=f32) + bs2_ref[0])
    sc = jnp.sum(s * ws3_ref[0], axis=1, keepdims=True) + bs3_ref[0]
    lane = lax.broadcasted_iota(jnp.int32, (1, _E), 1)
    w = jnp.sum(jnp.where(lane == ie, g_ref[...], 0.0), axis=1,
                keepdims=True)
    w3 = w[:, :, None]
    tr3 = tr.reshape(_BLK, _M, _OUT) * w3
    sc3 = sc.reshape(_BLK, _M, 1) * w3

    @pl.when(ie == 0)
    def _init():
        traj_ref[...] = jnp.zeros_like(traj_ref)
        sc_ref[...] = jnp.zeros_like(sc_ref)

    traj_ref[...] += tr3
    sc_ref[...] += sc3


def _experts(xb16, gates, w1cat, wt2b, wt3b, ws2b, ws3t,
             bt1, bt2, bt3, bs1, bs2, bs3):
    bt1r = bt1.reshape(_E, 1, _H1)
    bt2r = bt2.reshape(_E, 1, _H2)
    bt3r = bt3.reshape(_E, 1, _OUT)
    bs1r = bs1.reshape(_E, 1, _S1)
    bs2r = bs2.reshape(_E, 1, _S2)
    bs3r = bs3.reshape(_E, 1, 1)
    pere = lambda a, b: pl.BlockSpec((1, a, b), lambda i, e: (e, 0, 0))
    return pl.pallas_call(
        _experts_body,
        grid=(_B // _BLK, _E),
        in_specs=[
            pl.BlockSpec((_BLK, _M, _D), lambda i, e: (i, 0, 0)),
            pl.BlockSpec((_BLK, _E), lambda i, e: (i, 0)),
            pere(_D, _W1N),
            pere(_H1, _H2),
            pere(_H2, _OUT),
            pere(_S1, _S2),
            pere(1, _S2),
            pere(1, _H1),
            pere(1, _H2),
            pere(1, _OUT),
            pere(1, _S1),
            pere(1, _S2),
            pere(1, 1),
        ],
        out_specs=[
            pl.BlockSpec((_BLK, _M, _OUT), lambda i, e: (i, 0, 0)),
            pl.BlockSpec((_BLK, _M, 1), lambda i, e: (i, 0, 0)),
        ],
        out_shape=[
            jax.ShapeDtypeStruct((_B, _M, _OUT), jnp.float32),
            jax.ShapeDtypeStruct((_B, _M, 1), jnp.float32),
        ],
        compiler_params=pltpu.CompilerParams(
            dimension_semantics=("arbitrary", "arbitrary")),
    )(xb16, gates, w1cat, wt2b, wt3b, ws2b, ws3t,
      bt1r, bt2r, bt3r, bs1r, bs2r, bs3r)


def kernel(mode_features, rW1, rb1, rW2, rb2, rW3, rb3, Wt1, bt1, Wt2, bt2,
           Wt3, bt3, Ws1, bs1, Ws2, bs2, Ws3, bs3):
    logits, router_probs, aux = _router(mode_features, rW1, rb1, rW2, rb2,
                                        rW3, rb3)
    gates = _gates(logits)
    bf = jnp.bfloat16
    w1cat = jnp.concatenate([Wt1, Ws1], axis=2).astype(bf)
    ws3t = jnp.swapaxes(Ws3, 1, 2)
    traj, sc = _experts(mode_features, gates, w1cat, Wt2.astype(bf),
                        Wt3.astype(bf), Ws2.astype(bf), ws3t,
                        bt1, bt2, bt3, bs1, bs2, bs3)
    trajectories = traj.reshape(_B, _M, _T, 2)
    scores = sc.reshape(_B, _M)
    return trajectories, scores, aux.reshape(()), router_probs
